# baseline (speedup 1.0000x reference)
import functools
import math

import jax
import jax.numpy as jnp
from jax import lax
from jax.experimental import pallas as pl
from jax.experimental.pallas import tpu as pltpu
from jax.experimental.pallas import tpu_sc as plsc

_S = 64.0
_MARGIN = 0.5
_COS_M = math.cos(_MARGIN)
_SIN_M = math.sin(_MARGIN)

_NC = 2
_NS = 16
_NW = _NC * _NS
_L = 16

_CH = 2944
_PW = 100096
_NCHUNK = _PW // _CH
_RPW = 32

_GATHER_DNUMS = lax.GatherDimensionNumbers(
    offset_dims=(), collapsed_slice_dims=(0,), start_index_map=(0,))


def _sqrt16(a):
    s = 0.5 * (a + 1.0)
    for _ in range(20):
        s = 0.5 * (s + a / jnp.maximum(s, 1e-30))
    return s


def _splat16(v16, idx):
    i = jnp.broadcast_to(idx, (_L,))[:, None]
    return lax.gather(
        v16, i, dimension_numbers=_GATHER_DNUMS, slice_sizes=(1,),
        mode=lax.GatherScatterMode.PROMISE_IN_BOUNDS)


def _gather_margin(logits_hbm, lab_v, gat_v, cor_v, sem, base):
    lane_iota = lax.iota(jnp.int32, _L)
    for w in range(_RPW // _L):
        lab = lab_v[pl.ds(w * _L, _L)]
        safe = jnp.where(lab < 0, 0, lab)
        cs = [safe[l] for l in range(_L)]
        handles = []
        for l in range(_L):
            r = base + w * _L + l
            c = cs[l]
            handles.append(
                pltpu.async_copy(
                    logits_hbm.at[
                        pl.ds(pl.multiple_of(r & ~7, 8), 8),
                        pl.ds(pl.multiple_of(c & ~127, 128), 128),
                    ],
                    gat_v.at[l],
                    sem,
                )
            )
        for h in handles:
            h.wait()
        acc = jnp.zeros((_L,), jnp.float32)
        for l in range(_L):
            k = w * _L + l
            c = cs[l]
            cc0 = (c & 127) & ~15
            v = gat_v[l, k & 7, pl.ds(pl.multiple_of(cc0, 8), _L)]
            acc = jnp.where(lane_iota == l, _splat16(v, c & 15), acc)
        sin_theta = _sqrt16(jnp.maximum((1.0 - acc) * (1.0 + acc), 0.0))
        cor_v[pl.ds(w * _L, _L)] = (_COS_M * acc - _SIN_M * sin_theta) * _S


def _scale_chunk(bin_ref, bout_ref, c0, labs, cors, lane_iota):
    for r in range(8):
        def vb(j, _, r=r):
            o0 = pl.multiple_of(j * (8 * _L), _L)
            for u in range(8):
                o = pl.multiple_of(o0 + u * _L, _L)
                bout_ref[r, pl.ds(o, _L)] = bin_ref[r, pl.ds(o, _L)] * _S
            return 0
        lax.fori_loop(0, _CH // (8 * _L), vb, 0)
    for r in range(8):
        cloc = labs[r] - c0
        valid = (cloc >= 0) & (cloc < _CH)
        off = pl.multiple_of(jnp.clip(cloc & ~15, 0, _CH - _L), _L)
        lane = jnp.where(valid, cloc & 15, _L)
        v = bout_ref[r, pl.ds(off, _L)]
        bout_ref[r, pl.ds(off, _L)] = jnp.where(
            lane_iota == jnp.broadcast_to(lane, (_L,)), cors[r], v)


def _stream_tilerow(logits_hbm, out_hbm, r0, bufs, sems, labs, cors,
                    lane_iota):
    bin0, bin1, bout0, bout1 = bufs
    si0, si1, so0, so1 = sems
    src = logits_hbm.at[pl.ds(r0, 8)]
    dst = out_hbm.at[pl.ds(r0, 8)]

    def cp_in(c0, buf, sem):
        pltpu.async_copy(src.at[:, pl.ds(c0, _CH)], buf, sem)

    def cp_out(buf, c0, sem):
        pltpu.async_copy(buf, dst.at[:, pl.ds(c0, _CH)], sem)

    def wait_in(buf, sem):
        pltpu.make_async_copy(src.at[:, pl.ds(0, _CH)], buf, sem).wait()

    def wait_out(buf, sem):
        pltpu.make_async_copy(buf, dst.at[:, pl.ds(0, _CH)], sem).wait()

    cp_in(0, bin0, si0)
    wait_in(bin0, si0)
    cp_in(_CH, bin1, si1)
    _scale_chunk(bin0, bout0, 0, labs, cors, lane_iota)
    cp_out(bout0, 0, so0)
    wait_in(bin1, si1)
    cp_in(2 * _CH, bin0, si0)
    _scale_chunk(bin1, bout1, _CH, labs, cors, lane_iota)
    cp_out(bout1, _CH, so1)

    def pair(i, _):
        c0 = pl.multiple_of(i * (2 * _CH), _L)
        c1 = pl.multiple_of(c0 + _CH, _L)
        c2 = pl.multiple_of(
            jnp.minimum(c1 + _CH, (_NCHUNK - 1) * _CH), _L)
        wait_in(bin0, si0)
        cp_in(c1, bin1, si1)
        wait_out(bout0, so0)
        _scale_chunk(bin0, bout0, c0, labs, cors, lane_iota)
        cp_out(bout0, c0, so0)
        wait_in(bin1, si1)
        cp_in(c2, bin0, si0)
        wait_out(bout1, so1)
        _scale_chunk(bin1, bout1, c1, labs, cors, lane_iota)
        cp_out(bout1, c1, so1)
        return 0

    lax.fori_loop(1, _NCHUNK // 2, pair, 0)
    wait_in(bin0, si0)
    wait_out(bout0, so0)
    wait_out(bout1, so1)


def _sc_body(logits_hbm, lab_hbm, out_hbm, lab_v, cor_v, gat_v,
             bin0, bin1, bout0, bout1, gsem, si0, si1, so0, so1):
    wid = lax.axis_index("s") * _NC + lax.axis_index("c")
    base = wid * _RPW
    lane_iota = lax.iota(jnp.int32, _L)
    pltpu.sync_copy(lab_hbm.at[pl.ds(base, _RPW)], lab_v)
    _gather_margin(logits_hbm, lab_v, gat_v, cor_v, gsem, base)
    labs_all = []
    cors_all = []
    for h in range(_RPW // _L):
        labv = lab_v[pl.ds(h * _L, _L)]
        corv = cor_v[pl.ds(h * _L, _L)]
        labs_all += [labv[l] for l in range(_L)]
        cors_all += [_splat16(corv, l) for l in range(_L)]
    for tr in range(_RPW // 8):
        r0 = pl.multiple_of(base + tr * 8, 8)
        _stream_tilerow(
            logits_hbm, out_hbm, r0,
            (bin0, bin1, bout0, bout1), (si0, si1, so0, so1),
            labs_all[tr * 8:(tr + 1) * 8], cors_all[tr * 8:(tr + 1) * 8],
            lane_iota)


def kernel(logits, labels):
    rows, cols = logits.shape
    mesh = plsc.VectorSubcoreMesh(core_axis_name="c", subcore_axis_name="s")
    return pl.kernel(
        _sc_body,
        out_type=jax.ShapeDtypeStruct((rows, cols), jnp.float32),
        mesh=mesh,
        scratch_types=[
            pltpu.VMEM((_RPW,), jnp.int32),
            pltpu.VMEM((_RPW,), jnp.float32),
            pltpu.VMEM((_L, 8, 128), jnp.float32),
            pltpu.VMEM((8, _CH), jnp.float32),
            pltpu.VMEM((8, _CH), jnp.float32),
            pltpu.VMEM((8, _CH), jnp.float32),
            pltpu.VMEM((8, _CH), jnp.float32),
            pltpu.SemaphoreType.DMA,
            pltpu.SemaphoreType.DMA,
            pltpu.SemaphoreType.DMA,
            pltpu.SemaphoreType.DMA,
            pltpu.SemaphoreType.DMA,
        ],
    )(logits, labels)


# speedup vs baseline: 1.0403x; 1.0403x over previous
import functools
import math

import jax
import jax.numpy as jnp
from jax import lax
from jax.experimental import pallas as pl
from jax.experimental.pallas import tpu as pltpu
from jax.experimental.pallas import tpu_sc as plsc

_S = 64.0
_MARGIN = 0.5
_COS_M = math.cos(_MARGIN)
_SIN_M = math.sin(_MARGIN)

_RB = 16
_CB = 100000
_NBUF = 2

_NC = 2
_NS = 16
_NW = _NC * _NS
_L = 16


def _sqrt16(a):
    s = 0.5 * (a + 1.0)
    for _ in range(20):
        s = 0.5 * (s + a / jnp.maximum(s, 1e-30))
    return s


def _sc_margin_body(logits_hbm, lab_hbm, out_hbm, lab_v, gat_v, cor_v, sem, *, per_w):
    wid = lax.axis_index("s") * _NC + lax.axis_index("c")
    base = wid * per_w
    lane_iota = lax.iota(jnp.int32, _L)
    pltpu.sync_copy(lab_hbm.at[pl.ds(base, per_w)], lab_v)
    for w in range(per_w // _L):
        lab = lab_v[pl.ds(w * _L, _L)]
        safe = jnp.where(lab < 0, 0, lab)
        cs = [safe[l] for l in range(_L)]
        handles = []
        for l in range(_L):
            r = base + w * _L + l
            c = cs[l]
            handles.append(
                pltpu.async_copy(
                    logits_hbm.at[
                        pl.ds(pl.multiple_of(r & ~7, 8), 8),
                        pl.ds(pl.multiple_of(c & ~127, 128), 128),
                    ],
                    gat_v.at[l],
                    sem,
                )
            )
        for h in handles:
            h.wait()
        acc = jnp.zeros((_L,), jnp.float32)
        for l in range(_L):
            k = w * _L + l
            c = cs[l]
            cc0 = (c & 127) & ~15
            v = gat_v[l, k & 7, pl.ds(pl.multiple_of(cc0, 8), _L)]
            idx = jnp.broadcast_to(c & 15, (_L,))[:, None]
            t16 = lax.gather(
                v, idx,
                dimension_numbers=lax.GatherDimensionNumbers(
                    offset_dims=(), collapsed_slice_dims=(0,), start_index_map=(0,)),
                slice_sizes=(1,),
                mode=lax.GatherScatterMode.PROMISE_IN_BOUNDS,
            )
            acc = jnp.where(lane_iota == l, t16, acc)
        sin_theta = _sqrt16(jnp.maximum((1.0 - acc) * (1.0 + acc), 0.0))
        cor_v[pl.ds(w * _L, _L)] = (_COS_M * acc - _SIN_M * sin_theta) * _S
    pltpu.sync_copy(cor_v, out_hbm.at[pl.ds(base, per_w)])


def _sc_margin(logits, labels):
    rows = labels.shape[0]
    per_w = rows // _NW
    mesh = plsc.VectorSubcoreMesh(core_axis_name="c", subcore_axis_name="s")
    return pl.kernel(
        functools.partial(_sc_margin_body, per_w=per_w),
        out_type=jax.ShapeDtypeStruct((rows,), jnp.float32),
        mesh=mesh,
        scratch_types=[
            pltpu.VMEM((per_w,), jnp.int32),
            pltpu.VMEM((_L, 8, 128), jnp.float32),
            pltpu.VMEM((per_w,), jnp.float32),
            pltpu.SemaphoreType.DMA,
        ],
    )(logits, labels)


def _dense_body(lab_ref, cor_ref, x_ref, out_ref, *, cb):
    j = pl.program_id(1)
    x = x_ref[...]
    lab = lab_ref[...]
    cor = cor_ref[...]
    cols = j * cb + jax.lax.broadcasted_iota(jnp.int32, x.shape, 1)
    out_ref[...] = jnp.where(cols == lab, cor, x * _S)


def kernel(logits, labels):
    rows, cols = logits.shape
    corrected = _sc_margin(logits, labels)
    lab2 = labels.reshape(rows, 1)
    cor2 = corrected.reshape(rows, 1)
    grid = (rows // _RB, pl.cdiv(cols, _CB))
    return pl.pallas_call(
        functools.partial(_dense_body, cb=_CB),
        grid=grid,
        in_specs=[
            pl.BlockSpec((_RB, 1), lambda i, j: (i, 0)),
            pl.BlockSpec((_RB, 1), lambda i, j: (i, 0)),
            pl.BlockSpec((_RB, _CB), lambda i, j: (i, j),
                         pipeline_mode=pl.Buffered(buffer_count=_NBUF)),
        ],
        out_specs=pl.BlockSpec((_RB, _CB), lambda i, j: (i, j),
                               pipeline_mode=pl.Buffered(buffer_count=_NBUF)),
        out_shape=jax.ShapeDtypeStruct((rows, cols), jnp.float32),
        compiler_params=pltpu.CompilerParams(
            dimension_semantics=("parallel", "arbitrary")),
    )(lab2, cor2, logits)
